# baseline (speedup 1.0000x reference)
import jax
import jax.numpy as jnp
from jax.experimental import pallas as pl
from jax.experimental.pallas import tpu as pltpu

B = 128
D = 128
NL = 100000
CQ = 5000
N_CLS = NL + CQ
SCALAR = 30.0
N_PART = 7

T = 5000
NT_LUT = NL // T
NT_CQ = CQ // T
NT = NT_LUT + NT_CQ


def _oim_kernel(x_ref, lut_ref, cq_ref, st_ref, mask_ref, w_ref,
                out_hbm, loss_ref, buf, m_sc, s_sc, tg_sc, wa_sc, sem):
    i = pl.program_id(0)
    slot = jax.lax.rem(i, 2)

    @pl.when(i == 0)
    def _init():
        m_sc[:] = jnp.full((B, 1), -jnp.inf, jnp.float32)
        s_sc[:] = jnp.zeros((B, 1), jnp.float32)
        tg_sc[:] = jnp.zeros((B, 1), jnp.float32)
        wa_sc[:] = jnp.zeros((B, 1), jnp.float32)

    @pl.when(i >= 2)
    def _drain():
        pltpu.make_async_copy(
            buf.at[slot], out_hbm.at[:, i - 2], sem.at[slot]
        ).wait()

    dn = (((1,), (1,)), ((), ()))

    @pl.when(i < NT_LUT)
    def _lut():
        buf[slot] = jax.lax.dot_general(
            x_ref[:], lut_ref[:], dn,
            preferred_element_type=jnp.float32)[:, None, :] * SCALAR

    @pl.when(i >= NT_LUT)
    def _cq():
        buf[slot] = jax.lax.dot_general(
            x_ref[:], cq_ref[:], dn,
            preferred_element_type=jnp.float32)[:, None, :] * SCALAR

    logits = buf[slot][:, 0, :]
    pltpu.make_async_copy(
        buf.at[slot], out_hbm.at[:, i], sem.at[slot]
    ).start()

    tmax = jnp.max(logits, axis=1, keepdims=True)
    newm = jnp.maximum(m_sc[:], tmax)
    s_sc[:] = (s_sc[:] * jnp.exp(m_sc[:] - newm)
               + jnp.sum(jnp.exp(logits - newm), axis=1, keepdims=True))
    m_sc[:] = newm

    t_rel = st_ref[:] - i * T
    cols = jax.lax.broadcasted_iota(jnp.int32, (B, T), 1)
    hit = cols == t_rel
    tg_sc[:] += jnp.sum(jnp.where(hit, logits, 0.0), axis=1, keepdims=True)
    wrow = w_ref[0]
    wa_sc[:] += jnp.sum(jnp.where(hit, wrow, 0.0), axis=1, keepdims=True)

    @pl.when(i == NT - 1)
    def _fin():
        pltpu.make_async_copy(
            buf.at[1 - slot], out_hbm.at[:, i - 1], sem.at[1 - slot]
        ).wait()
        pltpu.make_async_copy(
            buf.at[slot], out_hbm.at[:, i], sem.at[slot]
        ).wait()
        lse = m_sc[:] + jnp.log(s_sc[:])
        nll = lse - tg_sc[:]
        wm = wa_sc[:] * mask_ref[:]
        num = jnp.sum(nll * wm)
        den = jnp.sum(wm)
        loss_ref[:] = (num / jnp.maximum(den, 1e-12)) * jnp.ones((1, 1), jnp.float32)


@jax.jit
def kernel(inputs, targets, pad_ratios, part_idx, lut, cq, weight):
    vis_part = jnp.ceil(N_PART * (1.0 - pad_ratios))
    invis = part_idx.astype(jnp.float32) > vis_part
    unlab = targets < 0
    t = jnp.where(unlab, 5555, targets)
    t = jnp.where(invis, 7777, t)
    new_t = jnp.where(invis, 5555, t)
    new_t = jnp.where(unlab, 5555, new_t)
    mask = (new_t != 5555).astype(jnp.float32)
    safe_t = jnp.clip(new_t, 0, N_CLS - 1)

    logits, loss = pl.pallas_call(
        _oim_kernel,
        grid=(NT,),
        in_specs=[
            pl.BlockSpec((B, D), lambda i: (0, 0)),
            pl.BlockSpec((T, D), lambda i: (jnp.minimum(i, NT_LUT - 1), 0)),
            pl.BlockSpec((T, D), lambda i: (jnp.maximum(i - NT_LUT, 0), 0)),
            pl.BlockSpec((B, 1), lambda i: (0, 0)),
            pl.BlockSpec((B, 1), lambda i: (0, 0)),
            pl.BlockSpec((1, 1, T), lambda i: (i, 0, 0)),
        ],
        out_specs=[
            pl.BlockSpec(memory_space=pl.ANY),
            pl.BlockSpec((1, 1), lambda i: (0, 0)),
        ],
        out_shape=(
            jax.ShapeDtypeStruct((B, NT, 1, T), jnp.float32),
            jax.ShapeDtypeStruct((1, 1), jnp.float32),
        ),
        scratch_shapes=[
            pltpu.VMEM((2, B, 1, T), jnp.float32),
            pltpu.VMEM((B, 1), jnp.float32),
            pltpu.VMEM((B, 1), jnp.float32),
            pltpu.VMEM((B, 1), jnp.float32),
            pltpu.VMEM((B, 1), jnp.float32),
            pltpu.SemaphoreType.DMA((2,)),
        ],
    )(inputs, lut, cq, safe_t[:, None], mask[:, None],
      weight.reshape(NT, 1, T))
    return loss[0, 0], logits.reshape(B, N_CLS)


# speedup vs baseline: 1.1383x; 1.1383x over previous
import jax
import jax.numpy as jnp
from jax.experimental import pallas as pl
from jax.experimental.pallas import tpu as pltpu

B = 128
D = 128
NL = 100000
CQ = 5000
N_CLS = NL + CQ
SCALAR = 30.0
N_PART = 7

T = 5000
NT_LUT = NL // T
NT_CQ = CQ // T
NT = NT_LUT + NT_CQ


def _oim_kernel(x_ref, lut_ref, cq_ref, st_ref, wm_ref,
                out_hbm, loss_ref, buf, m_sc, s_sc, tg_sc, sem):
    i = pl.program_id(0)
    slot = jax.lax.rem(i, 2)

    @pl.when(i == 0)
    def _init():
        m_sc[:] = jnp.full((B, 1), -jnp.inf, jnp.float32)
        s_sc[:] = jnp.zeros((B, 1), jnp.float32)
        tg_sc[:] = jnp.zeros((B, 1), jnp.float32)

    @pl.when(i >= 2)
    def _drain():
        pltpu.make_async_copy(
            buf.at[slot], out_hbm.at[:, i - 2, 0], sem.at[slot]
        ).wait()

    dn = (((1,), (1,)), ((), ()))

    @pl.when(i < NT_LUT)
    def _lut():
        buf[slot] = jax.lax.dot_general(
            x_ref[:], lut_ref[:], dn,
            preferred_element_type=jnp.float32) * SCALAR

    @pl.when(i >= NT_LUT)
    def _cq():
        buf[slot] = jax.lax.dot_general(
            x_ref[:], cq_ref[:], dn,
            preferred_element_type=jnp.float32) * SCALAR

    logits = buf[slot]
    pltpu.make_async_copy(
        buf.at[slot], out_hbm.at[:, i, 0], sem.at[slot]
    ).start()

    tmax = jnp.max(logits, axis=1, keepdims=True)
    newm = jnp.maximum(m_sc[:], tmax)
    s_sc[:] = (s_sc[:] * jnp.exp(m_sc[:] - newm)
               + jnp.sum(jnp.exp(logits - newm), axis=1, keepdims=True))
    m_sc[:] = newm

    t_rel = st_ref[:] - i * T
    cols = jax.lax.broadcasted_iota(jnp.int32, (B, T), 1)
    hit = cols == t_rel
    tg_sc[:] += jnp.sum(jnp.where(hit, logits, 0.0), axis=1, keepdims=True)

    @pl.when(i == NT - 1)
    def _fin():
        pltpu.make_async_copy(
            buf.at[1 - slot], out_hbm.at[:, i - 1, 0], sem.at[1 - slot]
        ).wait()
        pltpu.make_async_copy(
            buf.at[slot], out_hbm.at[:, i, 0], sem.at[slot]
        ).wait()
        lse = m_sc[:] + jnp.log(s_sc[:])
        nll = lse - tg_sc[:]
        wm = wm_ref[:]
        num = jnp.sum(nll * wm)
        den = jnp.sum(wm)
        loss_ref[:] = (num / jnp.maximum(den, 1e-12)) * jnp.ones((1, 1), jnp.float32)


@jax.jit
def kernel(inputs, targets, pad_ratios, part_idx, lut, cq, weight):
    vis_part = jnp.ceil(N_PART * (1.0 - pad_ratios))
    invis = part_idx.astype(jnp.float32) > vis_part
    unlab = targets < 0
    t = jnp.where(unlab, 5555, targets)
    t = jnp.where(invis, 7777, t)
    new_t = jnp.where(invis, 5555, t)
    new_t = jnp.where(unlab, 5555, new_t)
    mask = (new_t != 5555).astype(jnp.float32)
    safe_t = jnp.clip(new_t, 0, N_CLS - 1)
    wmask = weight[safe_t] * mask

    logits, loss = pl.pallas_call(
        _oim_kernel,
        grid=(NT,),
        in_specs=[
            pl.BlockSpec((B, D), lambda i: (0, 0)),
            pl.BlockSpec((T, D), lambda i: (jnp.minimum(i, NT_LUT - 1), 0)),
            pl.BlockSpec((T, D), lambda i: (jnp.maximum(i - NT_LUT, 0), 0)),
            pl.BlockSpec((B, 1), lambda i: (0, 0)),
            pl.BlockSpec((B, 1), lambda i: (0, 0)),
        ],
        out_specs=[
            pl.BlockSpec(memory_space=pl.ANY),
            pl.BlockSpec((1, 1), lambda i: (0, 0)),
        ],
        out_shape=(
            jax.ShapeDtypeStruct((B, NT, 1, T), jnp.float32),
            jax.ShapeDtypeStruct((1, 1), jnp.float32),
        ),
        scratch_shapes=[
            pltpu.VMEM((2, B, T), jnp.float32),
            pltpu.VMEM((B, 1), jnp.float32),
            pltpu.VMEM((B, 1), jnp.float32),
            pltpu.VMEM((B, 1), jnp.float32),
            pltpu.SemaphoreType.DMA((2,)),
        ],
    )(inputs, lut, cq, safe_t[:, None], wmask[:, None])
    return loss[0, 0], logits.reshape(B, N_CLS)


# speedup vs baseline: 1.9280x; 1.6938x over previous
import jax
import jax.numpy as jnp
from jax.experimental import pallas as pl
from jax.experimental.pallas import tpu as pltpu

B = 128
D = 128
NL = 100000
CQ = 5000
N_CLS = NL + CQ
SCALAR = 30.0
N_PART = 7

TL = 5120
NFULL = NL // TL
LUT_TAIL = NL - NFULL * TL
CQ_HEAD = TL - LUT_TAIL
CQ_TAIL = CQ - CQ_HEAD
NT = NFULL + 2


def _oim_kernel(x_ref, lut_ref, cq_ref, st_ref, wm_ref,
                out_ref, loss_ref, cq_sc, m_sc, s_sc, tg_sc):
    i = pl.program_id(0)

    @pl.when(i == 0)
    def _init():
        m_sc[:] = jnp.full((B, 1), -jnp.inf, jnp.float32)
        s_sc[:] = jnp.zeros((B, 1), jnp.float32)
        tg_sc[:] = jnp.zeros((B, 1), jnp.float32)

    dn = (((1,), (1,)), ((), ()))

    @pl.when(i < NFULL)
    def _lut():
        out_ref[:] = jax.lax.dot_general(
            x_ref[:], lut_ref[:], dn, preferred_element_type=jnp.float32) * SCALAR

    @pl.when(i == NFULL)
    def _straddle():
        lut_part = jax.lax.dot_general(
            x_ref[:], lut_ref[:], dn, preferred_element_type=jnp.float32) * SCALAR
        cq_sc[:] = jax.lax.dot_general(
            x_ref[:], cq_ref[:], dn, preferred_element_type=jnp.float32) * SCALAR
        out_ref[:, :LUT_TAIL] = lut_part[:, :LUT_TAIL]
        out_ref[:, LUT_TAIL:] = cq_sc[:, :CQ_HEAD]

    @pl.when(i == NFULL + 1)
    def _cq_tail():
        out_ref[:, :CQ_TAIL] = cq_sc[:, CQ_HEAD:]

    def _update(vals, width):
        tmax = jnp.max(vals, axis=1, keepdims=True)
        newm = jnp.maximum(m_sc[:], tmax)
        s_sc[:] = (s_sc[:] * jnp.exp(m_sc[:] - newm)
                   + jnp.sum(jnp.exp(vals - newm), axis=1, keepdims=True))
        m_sc[:] = newm
        t_rel = st_ref[:] - i * TL
        cols = jax.lax.broadcasted_iota(jnp.int32, (B, width), 1)
        hit = cols == t_rel
        tg_sc[:] += jnp.sum(jnp.where(hit, vals, 0.0), axis=1, keepdims=True)

    @pl.when(i < NT - 1)
    def _stats_full():
        _update(out_ref[:], TL)

    @pl.when(i == NT - 1)
    def _stats_tail():
        _update(out_ref[:, :CQ_TAIL], CQ_TAIL)
        lse = m_sc[:] + jnp.log(s_sc[:])
        nll = lse - tg_sc[:]
        wm = wm_ref[:]
        num = jnp.sum(nll * wm)
        den = jnp.sum(wm)
        loss_ref[:] = (num / jnp.maximum(den, 1e-12)) * jnp.ones((1, 1), jnp.float32)


@jax.jit
def kernel(inputs, targets, pad_ratios, part_idx, lut, cq, weight):
    vis_part = jnp.ceil(N_PART * (1.0 - pad_ratios))
    invis = part_idx.astype(jnp.float32) > vis_part
    unlab = targets < 0
    t = jnp.where(unlab, 5555, targets)
    t = jnp.where(invis, 7777, t)
    new_t = jnp.where(invis, 5555, t)
    new_t = jnp.where(unlab, 5555, new_t)
    mask = (new_t != 5555).astype(jnp.float32)
    safe_t = jnp.clip(new_t, 0, N_CLS - 1)
    wmask = weight[safe_t] * mask

    logits, loss = pl.pallas_call(
        _oim_kernel,
        grid=(NT,),
        in_specs=[
            pl.BlockSpec((B, D), lambda i: (0, 0)),
            pl.BlockSpec((TL, D), lambda i: (jnp.minimum(i, NFULL), 0)),
            pl.BlockSpec((CQ, D), lambda i: (0, 0)),
            pl.BlockSpec((B, 1), lambda i: (0, 0)),
            pl.BlockSpec((B, 1), lambda i: (0, 0)),
        ],
        out_specs=[
            pl.BlockSpec((B, TL), lambda i: (0, i)),
            pl.BlockSpec((1, 1), lambda i: (0, 0)),
        ],
        out_shape=(
            jax.ShapeDtypeStruct((B, N_CLS), jnp.float32),
            jax.ShapeDtypeStruct((1, 1), jnp.float32),
        ),
        scratch_shapes=[
            pltpu.VMEM((B, CQ), jnp.float32),
            pltpu.VMEM((B, 1), jnp.float32),
            pltpu.VMEM((B, 1), jnp.float32),
            pltpu.VMEM((B, 1), jnp.float32),
        ],
    )(inputs, lut, cq, safe_t[:, None], wmask[:, None])
    return loss[0, 0], logits


# speedup vs baseline: 2.0595x; 1.0682x over previous
import jax
import jax.numpy as jnp
from jax.experimental import pallas as pl
from jax.experimental.pallas import tpu as pltpu

B = 128
D = 128
NL = 100000
CQ = 5000
N_CLS = NL + CQ
SCALAR = 30.0
N_PART = 7

TL = 10240
NFULL = NL // TL
LUT_TAIL = NL - NFULL * TL
CQ_HEAD = TL - LUT_TAIL
CQ_TAIL = CQ - CQ_HEAD
NT = NFULL + 2


def _oim_kernel(x_ref, lut_ref, cq_ref, st_ref, wm_ref,
                out_ref, loss_ref, cq_sc, m_sc, s_sc, tg_sc):
    i = pl.program_id(0)

    @pl.when(i == 0)
    def _init():
        m_sc[:] = jnp.full((B, 1), -jnp.inf, jnp.float32)
        s_sc[:] = jnp.zeros((B, 1), jnp.float32)
        tg_sc[:] = jnp.zeros((B, 1), jnp.float32)

    dn = (((1,), (1,)), ((), ()))

    def _update(vals, width):
        tmax = jnp.max(vals, axis=1, keepdims=True)
        newm = jnp.maximum(m_sc[:], tmax)
        s_sc[:] = (s_sc[:] * jnp.exp(m_sc[:] - newm)
                   + jnp.sum(jnp.exp(vals - newm), axis=1, keepdims=True))
        m_sc[:] = newm
        t_rel = st_ref[:] - i * TL
        cols = jax.lax.broadcasted_iota(jnp.int32, (B, width), 1)
        hit = cols == t_rel
        tg_sc[:] += jnp.sum(jnp.where(hit, vals, 0.0), axis=1, keepdims=True)

    @pl.when(i < NFULL)
    def _lut():
        vals = jax.lax.dot_general(
            x_ref[:], lut_ref[:], dn, preferred_element_type=jnp.float32)
        out_ref[:] = vals
        _update(vals, TL)

    @pl.when(i == NFULL)
    def _straddle():
        lut_part = jax.lax.dot_general(
            x_ref[:], lut_ref[:], dn, preferred_element_type=jnp.float32)
        cq_sc[:] = jax.lax.dot_general(
            x_ref[:], cq_ref[:], dn, preferred_element_type=jnp.float32)
        out_ref[:, :LUT_TAIL] = lut_part[:, :LUT_TAIL]
        out_ref[:, LUT_TAIL:] = cq_sc[:, :CQ_HEAD]
        _update(out_ref[:], TL)

    @pl.when(i == NFULL + 1)
    def _cq_tail():
        vals = cq_sc[:, CQ_HEAD:]
        out_ref[:, :CQ_TAIL] = vals
        _update(vals, CQ_TAIL)
        lse = m_sc[:] + jnp.log(s_sc[:])
        nll = lse - tg_sc[:]
        wm = wm_ref[:]
        num = jnp.sum(nll * wm)
        den = jnp.sum(wm)
        loss_ref[:] = (num / jnp.maximum(den, 1e-12)) * jnp.ones((1, 1), jnp.float32)


@jax.jit
def kernel(inputs, targets, pad_ratios, part_idx, lut, cq, weight):
    vis_part = jnp.ceil(N_PART * (1.0 - pad_ratios))
    invis = part_idx.astype(jnp.float32) > vis_part
    unlab = targets < 0
    t = jnp.where(unlab, 5555, targets)
    t = jnp.where(invis, 7777, t)
    new_t = jnp.where(invis, 5555, t)
    new_t = jnp.where(unlab, 5555, new_t)
    mask = (new_t != 5555).astype(jnp.float32)
    safe_t = jnp.clip(new_t, 0, N_CLS - 1)
    wmask = weight[safe_t] * mask

    logits, loss = pl.pallas_call(
        _oim_kernel,
        grid=(NT,),
        in_specs=[
            pl.BlockSpec((B, D), lambda i: (0, 0)),
            pl.BlockSpec((TL, D), lambda i: (jnp.minimum(i, NFULL), 0)),
            pl.BlockSpec((CQ, D), lambda i: (0, 0)),
            pl.BlockSpec((B, 1), lambda i: (0, 0)),
            pl.BlockSpec((B, 1), lambda i: (0, 0)),
        ],
        out_specs=[
            pl.BlockSpec((B, TL), lambda i: (0, i)),
            pl.BlockSpec((1, 1), lambda i: (0, 0)),
        ],
        out_shape=(
            jax.ShapeDtypeStruct((B, N_CLS), jnp.float32),
            jax.ShapeDtypeStruct((1, 1), jnp.float32),
        ),
        scratch_shapes=[
            pltpu.VMEM((B, CQ), jnp.float32),
            pltpu.VMEM((B, 1), jnp.float32),
            pltpu.VMEM((B, 1), jnp.float32),
            pltpu.VMEM((B, 1), jnp.float32),
        ],
    )(inputs * SCALAR, lut, cq, safe_t[:, None], wmask[:, None])
    return loss[0, 0], logits


# speedup vs baseline: 2.0648x; 1.0025x over previous
import jax
import jax.numpy as jnp
from jax.experimental import pallas as pl
from jax.experimental.pallas import tpu as pltpu

B = 128
D = 128
NL = 100000
CQ = 5000
N_CLS = NL + CQ
SCALAR = 30.0
N_PART = 7

TL = 10240
NFULL = NL // TL
LUT_TAIL = NL - NFULL * TL
CQ_HEAD = TL - LUT_TAIL
CQ_TAIL = CQ - CQ_HEAD
NT = NFULL + 2


def _oim_kernel(x_ref, lut_ref, cq_ref, st_sm, st_vm, wm_ref, lut_any, cq_any,
                out_ref, loss_ref, cq_sc, m_sc, s_sc, rows8, gsem):
    i = pl.program_id(0)

    @pl.when(i == 0)
    def _init():
        m_sc[:] = jnp.full((B, 1), -jnp.inf, jnp.float32)
        s_sc[:] = jnp.zeros((B, 1), jnp.float32)
        for r in range(B):
            v = st_sm[r, 0]
            dst = rows8.at[pl.ds(r * 8, 8), :]

            @pl.when(v < NL)
            def _():
                base = (v // 8) * 8
                pltpu.make_async_copy(
                    lut_any.at[pl.ds(base, 8), :], dst, gsem).start()

            @pl.when(v >= NL)
            def _():
                base = ((v - NL) // 8) * 8
                pltpu.make_async_copy(
                    cq_any.at[pl.ds(base, 8), :], dst, gsem).start()

    dn = (((1,), (1,)), ((), ()))

    def _update(vals):
        tmax = jnp.max(vals, axis=1, keepdims=True)
        newm = jnp.maximum(m_sc[:], tmax)
        s_sc[:] = (s_sc[:] * jnp.exp(m_sc[:] - newm)
                   + jnp.sum(jnp.exp(vals - newm), axis=1, keepdims=True))
        m_sc[:] = newm

    @pl.when(i < NFULL)
    def _lut():
        vals = jax.lax.dot_general(
            x_ref[:], lut_ref[:], dn, preferred_element_type=jnp.float32)
        out_ref[:] = vals
        _update(vals)

    @pl.when(i == NFULL)
    def _straddle():
        lut_part = jax.lax.dot_general(
            x_ref[:], lut_ref[:], dn, preferred_element_type=jnp.float32)
        cq_sc[:] = jax.lax.dot_general(
            x_ref[:], cq_ref[:], dn, preferred_element_type=jnp.float32)
        out_ref[:, :LUT_TAIL] = lut_part[:, :LUT_TAIL]
        out_ref[:, LUT_TAIL:] = cq_sc[:, :CQ_HEAD]
        _update(out_ref[:])

    @pl.when(i == NT - 1)
    def _cq_tail():
        vals = cq_sc[:, CQ_HEAD:]
        out_ref[:, :CQ_TAIL] = vals
        _update(vals)
        for r in range(B):
            pltpu.make_async_copy(
                lut_any.at[pl.ds(0, 8), :], rows8.at[pl.ds(0, 8), :], gsem
            ).wait()
        rows3 = rows8[:].reshape(B, 8, D)
        sub = jax.lax.broadcasted_iota(jnp.int32, (B, 8), 1)
        sel = (sub == st_vm[:] % 8).astype(jnp.float32)[:, :, None]
        picked = jnp.sum(rows3 * sel, axis=1)
        lse = m_sc[:] + jnp.log(s_sc[:])
        wm = wm_ref[:]
        num = jnp.sum(lse * wm) - jnp.sum((x_ref[:] * wm) * picked)
        den = jnp.sum(wm)
        loss_ref[:] = (num / jnp.maximum(den, 1e-12)) * jnp.ones((1, 1), jnp.float32)


@jax.jit
def kernel(inputs, targets, pad_ratios, part_idx, lut, cq, weight):
    vis_part = jnp.ceil(N_PART * (1.0 - pad_ratios))
    invis = part_idx.astype(jnp.float32) > vis_part
    unlab = targets < 0
    t = jnp.where(unlab, 5555, targets)
    t = jnp.where(invis, 7777, t)
    new_t = jnp.where(invis, 5555, t)
    new_t = jnp.where(unlab, 5555, new_t)
    mask = (new_t != 5555).astype(jnp.float32)
    safe_t = jnp.clip(new_t, 0, N_CLS - 1)
    wmask = weight[safe_t] * mask

    logits, loss = pl.pallas_call(
        _oim_kernel,
        grid=(NT,),
        in_specs=[
            pl.BlockSpec((B, D), lambda i: (0, 0)),
            pl.BlockSpec((TL, D), lambda i: (jnp.minimum(i, NFULL), 0)),
            pl.BlockSpec((CQ, D), lambda i: (0, 0)),
            pl.BlockSpec(memory_space=pltpu.SMEM),
            pl.BlockSpec((B, 1), lambda i: (0, 0)),
            pl.BlockSpec((B, 1), lambda i: (0, 0)),
            pl.BlockSpec(memory_space=pl.ANY),
            pl.BlockSpec(memory_space=pl.ANY),
        ],
        out_specs=[
            pl.BlockSpec((B, TL), lambda i: (0, i)),
            pl.BlockSpec((1, 1), lambda i: (0, 0)),
        ],
        out_shape=(
            jax.ShapeDtypeStruct((B, N_CLS), jnp.float32),
            jax.ShapeDtypeStruct((1, 1), jnp.float32),
        ),
        scratch_shapes=[
            pltpu.VMEM((B, CQ), jnp.float32),
            pltpu.VMEM((B, 1), jnp.float32),
            pltpu.VMEM((B, 1), jnp.float32),
            pltpu.VMEM((B * 8, D), jnp.float32),
            pltpu.SemaphoreType.DMA,
        ],
    )(inputs * SCALAR, lut, cq, safe_t[:, None], safe_t[:, None],
      wmask[:, None], lut, cq)
    return loss[0, 0], logits
